# initial kernel scaffold (unmeasured)
import jax
import jax.numpy as jnp
from jax import lax
from jax.experimental import pallas as pl
from jax.experimental.pallas import tpu as pltpu

E = 8
EL = 4
C = 576
TJ = 512


def _a2a_exchange(t, cid):

    def body(src_ref, dst_ref, send_sem, recv_sem):
        my_x = lax.axis_index("x")
        my_y = lax.axis_index("y")
        my_z = lax.axis_index("z")
        nbr = (1 - my_x, my_y, my_z)

        barrier_sem = pltpu.get_barrier_semaphore()
        pl.semaphore_signal(
            barrier_sem, inc=1, device_id=nbr,
            device_id_type=pl.DeviceIdType.MESH,
        )
        pl.semaphore_wait(barrier_sem, 1)

        rdma = pltpu.make_async_remote_copy(
            src_ref=src_ref,
            dst_ref=dst_ref,
            send_sem=send_sem,
            recv_sem=recv_sem,
            device_id=nbr,
            device_id_type=pl.DeviceIdType.MESH,
        )
        rdma.start()
        rdma.wait()

    return pl.pallas_call(
        body,
        out_shape=jax.ShapeDtypeStruct(t.shape, t.dtype),
        in_specs=[pl.BlockSpec(memory_space=pltpu.ANY)],
        out_specs=pl.BlockSpec(memory_space=pltpu.ANY),
        scratch_shapes=[pltpu.SemaphoreType.DMA, pltpu.SemaphoreType.DMA],
        compiler_params=pltpu.CompilerParams(collective_id=cid),
    )(t)


def _expert_ffn(X, W1, W2):
    n_tok = X.shape[1]
    d_model = X.shape[2]
    d_ff = W1.shape[2]
    J = d_ff // TJ

    def body(x_ref, w1_ref, w2_ref, out_ref):
        j = pl.program_id(1)

        @pl.when(j == 0)
        def _():
            out_ref[...] = jnp.zeros_like(out_ref)

        h = jnp.maximum(
            jnp.dot(x_ref[0], w1_ref[0], preferred_element_type=jnp.float32),
            0.0,
        )
        out_ref[0] += jnp.dot(h, w2_ref[0], preferred_element_type=jnp.float32)

    return pl.pallas_call(
        body,
        grid=(EL, J),
        in_specs=[
            pl.BlockSpec((1, n_tok, d_model), lambda e, j: (e, 0, 0)),
            pl.BlockSpec((1, d_model, TJ), lambda e, j: (e, 0, j)),
            pl.BlockSpec((1, TJ, d_model), lambda e, j: (e, j, 0)),
        ],
        out_specs=pl.BlockSpec((1, n_tok, d_model), lambda e, j: (e, 0, 0)),
        out_shape=jax.ShapeDtypeStruct((EL, n_tok, d_model), jnp.float32),
        compiler_params=pltpu.CompilerParams(
            dimension_semantics=("arbitrary", "arbitrary"),
        ),
    )(X, W1, W2)


def kernel(x, assign, W1, W2):
    n, d = x.shape
    p = lax.axis_index("x")
    q = 1 - p

    order = jnp.argsort(assign)
    sorted_e = assign[order]
    first = jnp.searchsorted(sorted_e, jnp.arange(E))
    slot_sorted = jnp.arange(n, dtype=jnp.int32) - first[sorted_e]
    bidx_sorted = sorted_e * C + slot_sorted
    bidx_tok = jnp.zeros((n,), jnp.int32).at[order].set(bidx_sorted)

    s = jnp.arange(E * C, dtype=jnp.int32)
    pos = first[s // C] + s % C
    tok_for_slot = order[jnp.clip(pos, 0, n - 1)]
    xs = x[tok_for_slot].reshape(E, C, d)

    keep = lax.dynamic_slice(xs, (EL * p, 0, 0), (EL, C, d))
    send = lax.dynamic_slice(xs, (EL * q, 0, 0), (EL, C, d))

    xr = _a2a_exchange(send, cid=0)

    X = jnp.concatenate([keep, xr], axis=1)
    Y = _expert_ffn(X, W1, W2)

    yr = _a2a_exchange(Y[:, C:, :], cid=1)

    ob = jnp.zeros((E, C, d), jnp.float32)
    ob = lax.dynamic_update_slice(ob, Y[:, :C, :], (EL * p, 0, 0))
    ob = lax.dynamic_update_slice(ob, yr, (EL * q, 0, 0))
    return ob.reshape(E * C, d)[bidx_tok]


# baseline (device time: 2129972 ns/iter reference)
import jax
import jax.numpy as jnp
from jax import lax
from jax.experimental import pallas as pl
from jax.experimental.pallas import tpu as pltpu

E = 8
EL = 4
C = 576
TJ = 512
TM = 384


def _a2a_exchange(t, cid):

    def body(src_ref, dst_ref, send_sem, recv_sem):
        my_x = lax.axis_index("x")
        my_y = lax.axis_index("y")
        my_z = lax.axis_index("z")
        nbr = (1 - my_x, my_y, my_z)

        barrier_sem = pltpu.get_barrier_semaphore()
        pl.semaphore_signal(
            barrier_sem, inc=1, device_id=nbr,
            device_id_type=pl.DeviceIdType.MESH,
        )
        pl.semaphore_wait(barrier_sem, 1)

        rdma = pltpu.make_async_remote_copy(
            src_ref=src_ref,
            dst_ref=dst_ref,
            send_sem=send_sem,
            recv_sem=recv_sem,
            device_id=nbr,
            device_id_type=pl.DeviceIdType.MESH,
        )
        rdma.start()
        rdma.wait()

    return pl.pallas_call(
        body,
        out_shape=jax.ShapeDtypeStruct(t.shape, t.dtype),
        in_specs=[pl.BlockSpec(memory_space=pl.ANY)],
        out_specs=pl.BlockSpec(memory_space=pl.ANY),
        scratch_shapes=[pltpu.SemaphoreType.DMA, pltpu.SemaphoreType.DMA],
        compiler_params=pltpu.CompilerParams(collective_id=cid),
    )(t)


def _expert_ffn(X, W1, W2):
    n_tok = X.shape[1]
    d_model = X.shape[2]
    d_ff = W1.shape[2]
    J = d_ff // TJ
    M = n_tok // TM

    def body(x_ref, w1_ref, w2_ref, out_ref):
        j = pl.program_id(2)

        @pl.when(j == 0)
        def _():
            out_ref[...] = jnp.zeros_like(out_ref)

        h = jnp.maximum(
            jnp.dot(x_ref[0], w1_ref[0], preferred_element_type=jnp.float32),
            0.0,
        )
        out_ref[0] += jnp.dot(h, w2_ref[0], preferred_element_type=jnp.float32)

    return pl.pallas_call(
        body,
        grid=(EL, M, J),
        in_specs=[
            pl.BlockSpec((1, TM, d_model), lambda e, m, j: (e, m, 0)),
            pl.BlockSpec((1, d_model, TJ), lambda e, m, j: (e, 0, j)),
            pl.BlockSpec((1, TJ, d_model), lambda e, m, j: (e, j, 0)),
        ],
        out_specs=pl.BlockSpec((1, TM, d_model), lambda e, m, j: (e, m, 0)),
        out_shape=jax.ShapeDtypeStruct((EL, n_tok, d_model), jnp.float32),
        compiler_params=pltpu.CompilerParams(
            dimension_semantics=("arbitrary", "arbitrary", "arbitrary"),
        ),
    )(X, W1, W2)


def kernel(x, assign, W1, W2):
    n, d = x.shape
    p = lax.axis_index("x")
    q = 1 - p

    order = jnp.argsort(assign)
    sorted_e = assign[order]
    first = jnp.searchsorted(sorted_e, jnp.arange(E))
    slot_sorted = jnp.arange(n, dtype=jnp.int32) - first[sorted_e]
    bidx_sorted = sorted_e * C + slot_sorted
    bidx_tok = jnp.zeros((n,), jnp.int32).at[order].set(bidx_sorted)

    s = jnp.arange(E * C, dtype=jnp.int32)
    pos = first[s // C] + s % C
    tok_for_slot = order[jnp.clip(pos, 0, n - 1)]
    xs = x[tok_for_slot].reshape(E, C, d)

    keep = lax.dynamic_slice(xs, (EL * p, 0, 0), (EL, C, d))
    send = lax.dynamic_slice(xs, (EL * q, 0, 0), (EL, C, d))

    xr = _a2a_exchange(send, cid=0)

    X = jnp.concatenate([keep, xr], axis=1)
    Y = _expert_ffn(X, W1, W2)

    yr = _a2a_exchange(Y[:, C:, :], cid=1)

    ob = jnp.zeros((E, C, d), jnp.float32)
    ob = lax.dynamic_update_slice(ob, Y[:, :C, :], (EL * p, 0, 0))
    ob = lax.dynamic_update_slice(ob, yr, (EL * q, 0, 0))
    return ob.reshape(E * C, d)[bidx_tok]
